# Initial kernel scaffold; baseline (speedup 1.0000x reference)
import jax
import jax.numpy as jnp
from jax.experimental import pallas as pl


def kernel(x, table):
    raise NotImplementedError("write your pallas kernel here")


# speedup vs baseline: 16.8684x; 16.8684x over previous
import functools

import jax
import jax.numpy as jnp
from jax import lax
from jax.experimental import pallas as pl
from jax.experimental.pallas import tpu as pltpu
from jax.experimental.pallas import tpu_sc as plsc

_NC = 2
_NS = 16
_NW = _NC * _NS

_V = 32
_D = 64
_N = 4096 * 16 * 16
_CHUNK = 1024
_GATHERS = _CHUNK // 128
_PER_W = _N // _NW
_ITERS = _PER_W // _CHUNK


def _combine_body(t_ref, out_ref):
    t = t_ref[...]
    t0 = t[0:_V]
    t1 = t[_V:2 * _V]
    s01 = t0[:, None, :] + t1[None, :, :]
    out_ref[...] = s01.reshape(_V * _V, _D)


def _build_t01(table):
    return pl.pallas_call(
        _combine_body,
        out_shape=jax.ShapeDtypeStruct((_V * _V, _D), jnp.float32),
    )(table)


def _sc_body(iv01_hbm, iv2_hbm, t01_hbm, t2_hbm, out_hbm,
             idx01_v, idx2_v, rows_v, t01_sp, t2_sp, gsem):
    sid = lax.axis_index("s")
    wid = sid * _NC + lax.axis_index("c")

    r01 = (_V * _V) // _NS
    pltpu.sync_copy(t01_hbm.at[pl.ds(sid * r01, r01)],
                    t01_sp.at[pl.ds(sid * r01, r01)])
    r2 = _V // _NS
    pltpu.sync_copy(t2_hbm.at[pl.ds(sid * r2, r2)],
                    t2_sp.at[pl.ds(sid * r2, r2)])
    plsc.subcore_barrier()

    def chunk(it, carry):
        pos = (wid * _ITERS + it) * _CHUNK
        pltpu.sync_copy(iv01_hbm.at[pl.ds(pos, _CHUNK)], idx01_v)
        pltpu.sync_copy(iv2_hbm.at[pl.ds(pos, _CHUNK)], idx2_v)
        copies = [
            pltpu.async_copy(
                t01_sp.at[idx01_v.at[pl.ds(j * 128, 128)]], rows_v.at[j],
                gsem)
            for j in range(_GATHERS)
        ]
        for c_ in copies:
            c_.wait()
        adds = [
            pltpu.async_copy(
                t2_sp.at[idx2_v.at[pl.ds(j * 128, 128)]], rows_v.at[j],
                gsem, add=True)
            for j in range(_GATHERS)
        ]
        for c_ in adds:
            c_.wait()
        blk = (wid * _ITERS + it) * _GATHERS
        pltpu.sync_copy(rows_v, out_hbm.at[pl.ds(blk, _GATHERS)])
        return carry

    lax.fori_loop(0, _ITERS, chunk, 0)


def _sc_gather(iv01, iv2, t01, t2):
    mesh = plsc.VectorSubcoreMesh(core_axis_name="c", subcore_axis_name="s")
    f = functools.partial(
        pl.kernel,
        mesh=mesh,
        out_type=jax.ShapeDtypeStruct((_N // 128, 128, _D), jnp.float32),
        scratch_types=[
            pltpu.VMEM((_CHUNK,), jnp.int32),
            pltpu.VMEM((_CHUNK,), jnp.int32),
            pltpu.VMEM((_GATHERS, 128, _D), jnp.float32),
            pltpu.VMEM_SHARED((_V * _V, _D), jnp.float32),
            pltpu.VMEM_SHARED((_V, _D), jnp.float32),
            pltpu.SemaphoreType.DMA,
        ],
        compiler_params=pltpu.CompilerParams(use_tc_tiling_on_sc=False),
    )(_sc_body)
    return f(iv01, iv2, t01, t2)


def kernel(x, table):
    B, H, W, _ = x.shape
    iv01 = ((x[..., 0] << 5) | x[..., 1]).reshape(B * H * W)
    iv2 = x[..., 2].reshape(B * H * W)
    t01 = _build_t01(table)
    t2 = table[2 * _V:]
    out3 = _sc_gather(iv01, iv2, t01, t2)
    return out3.reshape(B, H, W, _D)


# speedup vs baseline: 21.8295x; 1.2941x over previous
import functools

import jax
import jax.numpy as jnp
from jax import lax
from jax.experimental import pallas as pl
from jax.experimental.pallas import tpu as pltpu
from jax.experimental.pallas import tpu_sc as plsc

_NC = 2
_NS = 16
_NW = _NC * _NS

_V = 32
_D = 64
_N = 4096 * 16 * 16
_CHUNK = 1024
_GATHERS = _CHUNK // 128
_PER_W = _N // _NW
_ITERS = _PER_W // _CHUNK


def _combine_body(t_ref, out_ref):
    t = t_ref[...]
    t0 = t[0:_V]
    t1 = t[_V:2 * _V]
    t2 = t[2 * _V:3 * _V]
    s01 = t0[:, None, :] + t1[None, :, :]
    s012 = s01[:, :, None, :] + t2[None, None, :, :]
    out_ref[...] = s012.reshape(_V * _V * _V, _D)


def _build_t012(table):
    return pl.pallas_call(
        _combine_body,
        out_shape=jax.ShapeDtypeStruct((_V * _V * _V, _D), jnp.float32),
    )(table)


def _sc_body(iv_hbm, t012_hbm, out_hbm, idx_v, rows_v, gsem):
    wid = lax.axis_index("s") * _NC + lax.axis_index("c")

    def chunk(it, carry):
        pos = (wid * _ITERS + it) * _CHUNK
        pltpu.sync_copy(iv_hbm.at[pl.ds(pos, _CHUNK)], idx_v)
        copies = [
            pltpu.async_copy(
                t012_hbm.at[idx_v.at[pl.ds(j * 128, 128)]], rows_v.at[j], gsem)
            for j in range(_GATHERS)
        ]
        for c_ in copies:
            c_.wait()
        blk = (wid * _ITERS + it) * _GATHERS
        pltpu.sync_copy(rows_v, out_hbm.at[pl.ds(blk, _GATHERS)])
        return carry

    lax.fori_loop(0, _ITERS, chunk, 0)


def _sc_gather(iv, t012):
    mesh = plsc.VectorSubcoreMesh(core_axis_name="c", subcore_axis_name="s")
    f = functools.partial(
        pl.kernel,
        mesh=mesh,
        out_type=jax.ShapeDtypeStruct((_N // 128, 128, _D), jnp.float32),
        scratch_types=[
            pltpu.VMEM((_CHUNK,), jnp.int32),
            pltpu.VMEM((_GATHERS, 128, _D), jnp.float32),
            pltpu.SemaphoreType.DMA,
        ],
        compiler_params=pltpu.CompilerParams(use_tc_tiling_on_sc=False),
    )(_sc_body)
    return f(iv, t012)


_BB = 128
_KB = 8


def _relayout_body(in_ref, out_ref):
    x = in_ref[...]
    for j in range(_KB):
        out_ref[2 * j] = x[:, j, 0:_D].T
        out_ref[2 * j + 1] = x[:, j, _D:2 * _D].T


def _relayout(v):
    B = 4096
    HW = 256
    grid = (B // _BB, (HW // 2) // _KB)
    return pl.pallas_call(
        _relayout_body,
        grid=grid,
        in_specs=[pl.BlockSpec((_BB, _KB, 128), lambda bi, ki: (bi, ki, 0))],
        out_specs=pl.BlockSpec((2 * _KB, _D, _BB), lambda bi, ki: (ki, 0, bi)),
        out_shape=jax.ShapeDtypeStruct((HW, _D, B), jnp.float32),
    )(v)


def kernel(x, table):
    B, H, W, _ = x.shape
    iv = (x[..., 0] << 10) | (x[..., 1] << 5) | x[..., 2]
    iv = iv.reshape(B * H * W)
    t012 = _build_t012(table)
    out3 = _sc_gather(iv, t012)
    v = out3.reshape(B, H * W // 2, 128)
    outT = _relayout(v)
    out4 = outT.reshape(H, W, _D, B)
    return jnp.transpose(out4, (3, 0, 1, 2))
